# initial kernel scaffold (unmeasured)
import functools

import jax
import jax.numpy as jnp
from jax import lax
from jax.experimental import pallas as pl
from jax.experimental.pallas import tpu as pltpu

N_DEV = 8
SQ = 256
SKV = 2048
HQ = 8
DH = 128
DM = 1024
SCALE = 0.08838834764831843


def _attn_partial(c, xc, wq_ref, k_ref, v_ref, wo_ref):
    q = lax.dot_general(
        xc, wq_ref[...], (((1,), (0,)), ((), ())),
        preferred_element_type=jnp.float32,
    )

    rows = lax.broadcasted_iota(jnp.int32, (SQ, SKV), 0) + c * SQ
    cols = lax.broadcasted_iota(jnp.int32, (SQ, SKV), 1)
    keep = (cols // 64) <= (rows // 64)

    def head_body(h, acc):
        qh = lax.dynamic_slice(q, (0, h * DH), (SQ, DH))
        s = lax.dot_general(
            qh, k_ref[h], (((1,), (1,)), ((), ())),
            preferred_element_type=jnp.float32,
        ) * SCALE
        s = jnp.where(keep, s, -1e9)
        m = jnp.max(s, axis=1, keepdims=True)
        w = jnp.exp(s - m)
        w = w / jnp.sum(w, axis=1, keepdims=True)
        ctx = lax.dot_general(
            w, v_ref[h], (((1,), (0,)), ((), ())),
            preferred_element_type=jnp.float32,
        )
        woh = wo_ref[pl.ds(h * DH, DH), :]
        return acc + lax.dot_general(
            ctx, woh, (((1,), (0,)), ((), ())),
            preferred_element_type=jnp.float32,
        )

    return lax.fori_loop(0, HQ, head_body, jnp.zeros((SQ, DM), jnp.float32))


def _body(x_ref, wq_ref, k_ref, v_ref, wo_ref, out_ref,
          xbuf, rsbuf, ag_send, ag_recv, rs_send, rs_recv):
    i = lax.axis_index("i")
    right = lax.rem(i + 1, N_DEV)

    xbuf[0] = x_ref[...]
    partial_own = _attn_partial(i, x_ref[...], wq_ref, k_ref, v_ref, wo_ref)

    for t in range(N_DEV - 1):
        ag = pltpu.make_async_remote_copy(
            src_ref=xbuf.at[t],
            dst_ref=xbuf.at[t + 1],
            send_sem=ag_send.at[t],
            recv_sem=ag_recv.at[t],
            device_id=(right,),
            device_id_type=pl.DeviceIdType.MESH,
        )
        ag.start()
        ag.wait()

        c_r = lax.rem(i - (t + 1) + N_DEV, N_DEV)
        pc = _attn_partial(c_r, xbuf[t + 1], wq_ref, k_ref, v_ref, wo_ref)

        slot = N_DEV - 1 if t == 0 else t - 1
        if t == 0:
            rsbuf[slot] = pc
        else:
            rsbuf[slot] = rsbuf[slot] + pc
        rs = pltpu.make_async_remote_copy(
            src_ref=rsbuf.at[slot],
            dst_ref=rsbuf.at[t],
            send_sem=rs_send.at[t],
            recv_sem=rs_recv.at[t],
            device_id=(right,),
            device_id_type=pl.DeviceIdType.MESH,
        )
        rs.start()
        rs.wait()

    out_ref[...] = rsbuf[N_DEV - 2] + partial_own


def kernel(x, Wq, K_ext, V_ext, Wo):
    i = lax.axis_index("i")
    k_sl = lax.dynamic_slice(K_ext, (0, 0, i * HQ, 0), (1, SKV, HQ, DH))[0]
    v_sl = lax.dynamic_slice(V_ext, (0, 0, i * HQ, 0), (1, SKV, HQ, DH))[0]
    k_hm = jnp.transpose(k_sl, (1, 0, 2))
    v_hm = jnp.transpose(v_sl, (1, 0, 2))

    out = pl.pallas_call(
        _body,
        out_shape=jax.ShapeDtypeStruct((SQ, DM), jnp.float32),
        in_specs=[pl.BlockSpec(memory_space=pltpu.VMEM)] * 5,
        out_specs=pl.BlockSpec(memory_space=pltpu.VMEM),
        scratch_shapes=[
            pltpu.VMEM((N_DEV, SQ, DM), jnp.float32),
            pltpu.VMEM((N_DEV, SQ, DM), jnp.float32),
            pltpu.SemaphoreType.DMA((N_DEV - 1,)),
            pltpu.SemaphoreType.DMA((N_DEV - 1,)),
            pltpu.SemaphoreType.DMA((N_DEV - 1,)),
            pltpu.SemaphoreType.DMA((N_DEV - 1,)),
        ],
        compiler_params=pltpu.CompilerParams(collective_id=0),
    )(x[0], Wq, k_hm, v_hm, Wo)
    return out.reshape(1, SQ, DM)


# baseline (device time: 343179 ns/iter reference)
import functools

import jax
import jax.numpy as jnp
from jax import lax
from jax.experimental import pallas as pl
from jax.experimental.pallas import tpu as pltpu

N_DEV = 8
SQ = 256
SKV = 2048
HQ = 8
DH = 128
DM = 1024
SCALE = 0.08838834764831843


def _attn_partial(c, xc, wq_ref, k_ref, v_ref, wo_ref):
    q = lax.dot_general(
        xc, wq_ref[...], (((1,), (0,)), ((), ())),
        preferred_element_type=jnp.float32,
    )

    rows = lax.broadcasted_iota(jnp.int32, (SQ, SKV), 0) + c * SQ
    cols = lax.broadcasted_iota(jnp.int32, (SQ, SKV), 1)
    keep = (cols // 64) <= (rows // 64)

    acc = jnp.zeros((SQ, DM), jnp.float32)
    for h in range(HQ):
        qh = q[:, h * DH:(h + 1) * DH]
        s = lax.dot_general(
            qh, k_ref[h], (((1,), (1,)), ((), ())),
            preferred_element_type=jnp.float32,
        ) * SCALE
        s = jnp.where(keep, s, -1e9)
        m = jnp.max(s, axis=1, keepdims=True)
        w = jnp.exp(s - m)
        w = w / jnp.sum(w, axis=1, keepdims=True)
        ctx = lax.dot_general(
            w, v_ref[h], (((1,), (0,)), ((), ())),
            preferred_element_type=jnp.float32,
        )
        woh = wo_ref[h * DH:(h + 1) * DH, :]
        acc = acc + lax.dot_general(
            ctx, woh, (((1,), (0,)), ((), ())),
            preferred_element_type=jnp.float32,
        )
    return acc


def _body(x_ref, wq_ref, k_ref, v_ref, wo_ref, out_ref,
          xbuf, rsbuf, ag_send, ag_recv, rs_send, rs_recv):
    i = lax.axis_index("i")
    right = lax.rem(i + 1, N_DEV)

    xbuf[0] = x_ref[...]
    partial_own = _attn_partial(i, x_ref[...], wq_ref, k_ref, v_ref, wo_ref)

    for t in range(N_DEV - 1):
        ag = pltpu.make_async_remote_copy(
            src_ref=xbuf.at[t],
            dst_ref=xbuf.at[t + 1],
            send_sem=ag_send.at[t],
            recv_sem=ag_recv.at[t],
            device_id=(right,),
            device_id_type=pl.DeviceIdType.MESH,
        )
        ag.start()
        ag.wait()

        c_r = lax.rem(i - (t + 1) + N_DEV, N_DEV)
        pc = _attn_partial(c_r, xbuf[t + 1], wq_ref, k_ref, v_ref, wo_ref)

        slot = N_DEV - 1 if t == 0 else t - 1
        if t == 0:
            rsbuf[slot] = pc
        else:
            rsbuf[slot] = rsbuf[slot] + pc
        rs = pltpu.make_async_remote_copy(
            src_ref=rsbuf.at[slot],
            dst_ref=rsbuf.at[t],
            send_sem=rs_send.at[t],
            recv_sem=rs_recv.at[t],
            device_id=(right,),
            device_id_type=pl.DeviceIdType.MESH,
        )
        rs.start()
        rs.wait()

    out_ref[...] = rsbuf[N_DEV - 2] + partial_own


def kernel(x, Wq, K_ext, V_ext, Wo):
    i = lax.axis_index("i")
    k_sl = lax.dynamic_slice(K_ext, (0, 0, i * HQ, 0), (1, SKV, HQ, DH))[0]
    v_sl = lax.dynamic_slice(V_ext, (0, 0, i * HQ, 0), (1, SKV, HQ, DH))[0]
    k_hm = jnp.transpose(k_sl, (1, 0, 2))
    v_hm = jnp.transpose(v_sl, (1, 0, 2))

    out = pl.pallas_call(
        _body,
        out_shape=jax.ShapeDtypeStruct((SQ, DM), jnp.float32),
        in_specs=[pl.BlockSpec(memory_space=pltpu.VMEM)] * 5,
        out_specs=pl.BlockSpec(memory_space=pltpu.VMEM),
        scratch_shapes=[
            pltpu.VMEM((N_DEV, SQ, DM), jnp.float32),
            pltpu.VMEM((N_DEV, SQ, DM), jnp.float32),
            pltpu.SemaphoreType.DMA((N_DEV - 1,)),
            pltpu.SemaphoreType.DMA((N_DEV - 1,)),
            pltpu.SemaphoreType.DMA((N_DEV - 1,)),
            pltpu.SemaphoreType.DMA((N_DEV - 1,)),
        ],
    )(x[0], Wq, k_hm, v_hm, Wo)
    return out.reshape(1, SQ, DM)


# device time: 208812 ns/iter; 1.6435x vs baseline; 1.6435x over previous
import functools

import jax
import jax.numpy as jnp
from jax import lax
from jax.experimental import pallas as pl
from jax.experimental.pallas import tpu as pltpu

N_DEV = 8
SQ = 256
SKV = 2048
HQ = 8
DH = 128
DM = 1024
SCALE = 0.08838834764831843


def _attn_partial(c, xc, wq_ref, k_ref, v_ref, wo_ref):
    q = lax.dot_general(
        xc, wq_ref[...], (((1,), (0,)), ((), ())),
        preferred_element_type=jnp.float32,
    )

    rows = lax.broadcasted_iota(jnp.int32, (SQ, SKV), 0) + c * SQ
    cols = lax.broadcasted_iota(jnp.int32, (SQ, SKV), 1)
    keep = (cols // 64) <= (rows // 64)

    acc = jnp.zeros((SQ, DM), jnp.float32)
    for h in range(HQ):
        qh = q[:, h * DH:(h + 1) * DH]
        s = lax.dot_general(
            qh, k_ref[h], (((1,), (1,)), ((), ())),
            preferred_element_type=jnp.float32,
        ) * SCALE
        s = jnp.where(keep, s, -1e9)
        m = jnp.max(s, axis=1, keepdims=True)
        w = jnp.exp(s - m)
        w = w / jnp.sum(w, axis=1, keepdims=True)
        ctx = lax.dot_general(
            w, v_ref[h], (((1,), (0,)), ((), ())),
            preferred_element_type=jnp.float32,
        )
        woh = wo_ref[h * DH:(h + 1) * DH, :]
        acc = acc + lax.dot_general(
            ctx, woh, (((1,), (0,)), ((), ())),
            preferred_element_type=jnp.float32,
        )
    return acc


def _body(x_ref, wq_ref, k_ref, v_ref, wo_ref, out_ref,
          xbuf, rsbuf, ag_send, ag_recv, rs_send, rs_recv):
    i = lax.axis_index("i")
    right = lax.rem(i + 1, N_DEV)

    def ag_rdma(t):
        return pltpu.make_async_remote_copy(
            src_ref=xbuf.at[t],
            dst_ref=xbuf.at[t + 1],
            send_sem=ag_send.at[t],
            recv_sem=ag_recv.at[t],
            device_id=(right,),
            device_id_type=pl.DeviceIdType.MESH,
        )

    def rs_rdma(t):
        slot = N_DEV - 1 if t == 0 else t - 1
        return pltpu.make_async_remote_copy(
            src_ref=rsbuf.at[slot],
            dst_ref=rsbuf.at[t],
            send_sem=rs_send.at[t],
            recv_sem=rs_recv.at[t],
            device_id=(right,),
            device_id_type=pl.DeviceIdType.MESH,
        )

    xbuf[0] = x_ref[...]
    ag_rdma(0).start()
    partial_own = _attn_partial(i, x_ref[...], wq_ref, k_ref, v_ref, wo_ref)

    for t in range(N_DEV - 1):
        ag_rdma(t).wait_recv()
        if t + 1 < N_DEV - 1:
            ag_rdma(t + 1).start()

        c_r = lax.rem(i - (t + 1) + N_DEV, N_DEV)
        pc = _attn_partial(c_r, xbuf[t + 1], wq_ref, k_ref, v_ref, wo_ref)

        if t == 0:
            rsbuf[N_DEV - 1] = pc
        else:
            rs_rdma(t - 1).wait_recv()
            rsbuf[t - 1] = rsbuf[t - 1] + pc
        rs_rdma(t).start()

    rs_rdma(N_DEV - 2).wait_recv()
    out_ref[...] = rsbuf[N_DEV - 2] + partial_own

    for t in range(N_DEV - 1):
        ag_rdma(t).wait_send()
        rs_rdma(t).wait_send()


def kernel(x, Wq, K_ext, V_ext, Wo):
    i = lax.axis_index("i")
    k_sl = lax.dynamic_slice(K_ext, (0, 0, i * HQ, 0), (1, SKV, HQ, DH))[0]
    v_sl = lax.dynamic_slice(V_ext, (0, 0, i * HQ, 0), (1, SKV, HQ, DH))[0]
    k_hm = jnp.transpose(k_sl, (1, 0, 2))
    v_hm = jnp.transpose(v_sl, (1, 0, 2))

    out = pl.pallas_call(
        _body,
        out_shape=jax.ShapeDtypeStruct((SQ, DM), jnp.float32),
        in_specs=[pl.BlockSpec(memory_space=pltpu.VMEM)] * 5,
        out_specs=pl.BlockSpec(memory_space=pltpu.VMEM),
        scratch_shapes=[
            pltpu.VMEM((N_DEV, SQ, DM), jnp.float32),
            pltpu.VMEM((N_DEV, SQ, DM), jnp.float32),
            pltpu.SemaphoreType.DMA((N_DEV - 1,)),
            pltpu.SemaphoreType.DMA((N_DEV - 1,)),
            pltpu.SemaphoreType.DMA((N_DEV - 1,)),
            pltpu.SemaphoreType.DMA((N_DEV - 1,)),
        ],
    )(x[0], Wq, k_hm, v_hm, Wo)
    return out.reshape(1, SQ, DM)
